# baseline (speedup 1.0000x reference)
import jax
import jax.numpy as jnp
from jax import lax
from jax.experimental import pallas as pl
from jax.experimental.pallas import tpu as pltpu
from jax.experimental.pallas import tpu_sc as plsc

VOCAB = 1000000
EMBED_DIM = 64
PAD_DIM = 128
BATCH = 4096
SEQ = 200

NUM_CORES = 2
NUM_SUBCORES = 16
NUM_WORKERS = NUM_CORES * NUM_SUBCORES
IDX_PER_WORKER = BATCH * SEQ // NUM_WORKERS
ROWS_PER_WORKER = BATCH // NUM_WORKERS
BLOCK = 128
NUM_BLOCKS = IDX_PER_WORKER // BLOCK


def _sc_pool_kernel(idx_hbm, scat_hbm, table_hbm, out_hbm,
                    idx_v, scat_v, rows_v, acc_sh, sems):
    cid = lax.axis_index("c")
    sid = lax.axis_index("s")
    wid = sid * NUM_CORES + cid

    pltpu.sync_copy(idx_hbm.at[wid], idx_v)
    pltpu.sync_copy(scat_hbm.at[sid], scat_v)

    zeros16 = jnp.zeros((16,), jnp.float32)

    @pl.loop(0, BLOCK)
    def _(r):
        @pl.loop(0, PAD_DIM, step=16)
        def _(c):
            rows_v[0, r, pl.ds(c, 16)] = zeros16

    pltpu.sync_copy(rows_v.at[0],
                    acc_sh.at[pl.ds(sid * ROWS_PER_WORKER, ROWS_PER_WORKER)])

    def start_gather(b, slot):
        pltpu.make_async_copy(
            table_hbm.at[idx_v.at[b]], rows_v.at[slot], sems.at[slot]
        ).start()

    def finish_gather(b, slot):
        pltpu.make_async_copy(
            table_hbm.at[idx_v.at[b]], rows_v.at[slot], sems.at[slot]
        ).wait()

    def scatter_add(b, slot):
        pltpu.sync_copy(rows_v.at[slot], acc_sh.at[scat_v.at[b]], add=True)

    start_gather(0, 0)

    @pl.loop(0, NUM_BLOCKS - 1, step=2)
    def _(b):
        finish_gather(b, 0)
        start_gather(b + 1, 1)
        scatter_add(b, 0)
        finish_gather(b + 1, 1)

        @pl.when(b + 2 < NUM_BLOCKS)
        def _():
            start_gather(b + 2, 0)

        scatter_add(b + 1, 1)

    pltpu.sync_copy(acc_sh.at[pl.ds(sid * ROWS_PER_WORKER, ROWS_PER_WORKER)],
                    out_hbm.at[pl.ds(wid * ROWS_PER_WORKER, ROWS_PER_WORKER)])


@jax.jit
def _sc_pool(input_text, table):
    idx = input_text.reshape(NUM_WORKERS, NUM_BLOCKS, BLOCK)
    base = (jnp.arange(IDX_PER_WORKER, dtype=jnp.int32) // SEQ).reshape(
        1, NUM_BLOCKS, BLOCK)
    offs = (jnp.arange(NUM_SUBCORES, dtype=jnp.int32)
            * ROWS_PER_WORKER).reshape(NUM_SUBCORES, 1, 1)
    scat = base + offs

    table_p = jnp.pad(table, ((0, 0), (0, PAD_DIM - EMBED_DIM)))

    mesh = plsc.VectorSubcoreMesh(core_axis_name="c", subcore_axis_name="s")
    kern = pl.kernel(
        _sc_pool_kernel,
        out_type=jax.ShapeDtypeStruct((BATCH, PAD_DIM), jnp.float32),
        mesh=mesh,
        compiler_params=pltpu.CompilerParams(use_tc_tiling_on_sc=False),
        scratch_types=[
            pltpu.VMEM((NUM_BLOCKS, BLOCK), jnp.int32),
            pltpu.VMEM((NUM_BLOCKS, BLOCK), jnp.int32),
            pltpu.VMEM((2, BLOCK, PAD_DIM), jnp.float32),
            pltpu.VMEM_SHARED((NUM_SUBCORES * ROWS_PER_WORKER, PAD_DIM),
                              jnp.float32),
            pltpu.SemaphoreType.DMA((2,)),
        ],
    )
    return kern(idx, scat, table_p)


def _mlp_kernel(x_ref, len_ref, w1_ref, b1_ref, w2_ref, b2_ref, out_ref):
    x = x_ref[:, :EMBED_DIM] / len_ref[...]
    h = jnp.maximum(
        jnp.dot(x, w1_ref[...], preferred_element_type=jnp.float32)
        + b1_ref[...], 0.0)
    out_ref[...] = (
        jnp.dot(h, w2_ref[...], preferred_element_type=jnp.float32)
        + b2_ref[...])


@jax.jit
def _mlp(pooled, text_len, W1, b1, W2, b2):
    bm = 512
    n_hidden = W1.shape[1]
    n_classes = W2.shape[1]
    grid = (BATCH // bm,)
    return pl.pallas_call(
        _mlp_kernel,
        grid=grid,
        in_specs=[
            pl.BlockSpec((bm, PAD_DIM), lambda i: (i, 0)),
            pl.BlockSpec((bm, 1), lambda i: (i, 0)),
            pl.BlockSpec((EMBED_DIM, n_hidden), lambda i: (0, 0)),
            pl.BlockSpec((1, n_hidden), lambda i: (0, 0)),
            pl.BlockSpec((n_hidden, n_classes), lambda i: (0, 0)),
            pl.BlockSpec((1, n_classes), lambda i: (0, 0)),
        ],
        out_specs=pl.BlockSpec((bm, n_classes), lambda i: (i, 0)),
        out_shape=jax.ShapeDtypeStruct((BATCH, n_classes), jnp.float32),
    )(pooled, text_len.reshape(BATCH, 1), W1, b1.reshape(1, n_hidden),
      W2, b2.reshape(1, n_classes))


def kernel(input_text, text_len, table, W1, b1, W2, b2):
    pooled = _sc_pool(input_text, table)
    return _mlp(pooled, text_len, W1, b1, W2, b2)


# speedup vs baseline: 1.1231x; 1.1231x over previous
import jax
import jax.numpy as jnp
from jax import lax
from jax.experimental import pallas as pl
from jax.experimental.pallas import tpu as pltpu
from jax.experimental.pallas import tpu_sc as plsc

VOCAB = 1000000
EMBED_DIM = 64
PAD_DIM = 128
BATCH = 4096
SEQ = 200

NUM_CORES = 2
NUM_SUBCORES = 16
NUM_WORKERS = NUM_CORES * NUM_SUBCORES
IDX_PER_WORKER = BATCH * SEQ // NUM_WORKERS
ROWS_PER_WORKER = BATCH // NUM_WORKERS
BLOCK = 128
NUM_BLOCKS = IDX_PER_WORKER // BLOCK

CHUNK = 1000
NUM_CHUNKS = VOCAB // CHUNK
CHUNKS_PER_WORKER = 32


def _sc_pool_kernel(idx_hbm, scat_hbm, table_hbm, out_hbm,
                    idx_v, scat_v, rows_v, acc_sh, gsems, ssems):
    cid = lax.axis_index("c")
    sid = lax.axis_index("s")
    wid = sid * NUM_CORES + cid

    pltpu.sync_copy(idx_hbm.at[wid], idx_v)
    pltpu.sync_copy(scat_hbm.at[sid], scat_v)

    zeros16 = jnp.zeros((16,), jnp.float32)

    @pl.loop(0, BLOCK)
    def _(r):
        @pl.loop(0, EMBED_DIM, step=16)
        def _(c):
            rows_v[0, r, pl.ds(c, 16)] = zeros16

    pltpu.sync_copy(rows_v.at[0],
                    acc_sh.at[pl.ds(sid * ROWS_PER_WORKER, ROWS_PER_WORKER)])

    def gather(b, slot):
        return pltpu.make_async_copy(
            table_hbm.at[idx_v.at[b]], rows_v.at[slot], gsems.at[slot])

    def scatter_start(b, slot):
        pltpu.async_copy(
            rows_v.at[slot], acc_sh.at[scat_v.at[b]], ssems.at[slot],
            add=True)

    def scatter_wait(b, slot):
        pltpu.make_async_copy(
            rows_v.at[slot], acc_sh.at[scat_v.at[b]], ssems.at[slot]).wait()

    gather(0, 0).start()
    gather(1, 1).start()

    @pl.loop(0, NUM_BLOCKS, step=2)
    def _(b):
        gather(b, 0).wait()
        scatter_start(b, 0)
        gather(b + 1, 1).wait()
        scatter_start(b + 1, 1)
        scatter_wait(b, 0)

        @pl.when(b + 2 < NUM_BLOCKS)
        def _():
            gather(b + 2, 0).start()

        scatter_wait(b + 1, 1)

        @pl.when(b + 3 < NUM_BLOCKS)
        def _():
            gather(b + 3, 1).start()

    pltpu.sync_copy(acc_sh.at[pl.ds(sid * ROWS_PER_WORKER, ROWS_PER_WORKER)],
                    out_hbm.at[pl.ds(wid * ROWS_PER_WORKER, ROWS_PER_WORKER)])


@jax.jit
def _sc_pool(input_text, table):
    idx = (input_text.reshape(NUM_WORKERS, NUM_BLOCKS, BLOCK)) * 2
    base = (jnp.arange(IDX_PER_WORKER, dtype=jnp.int32) // SEQ).reshape(
        1, NUM_BLOCKS, BLOCK)
    offs = (jnp.arange(NUM_SUBCORES, dtype=jnp.int32)
            * ROWS_PER_WORKER).reshape(NUM_SUBCORES, 1, 1)
    scat = base + offs

    mesh = plsc.VectorSubcoreMesh(core_axis_name="c", subcore_axis_name="s")

    table_lin = jnp.pad(
        table, ((0, 0), (0, PAD_DIM - EMBED_DIM))).reshape(
            2 * VOCAB, EMBED_DIM)

    pool = pl.kernel(
        _sc_pool_kernel,
        out_type=jax.ShapeDtypeStruct((BATCH, EMBED_DIM), jnp.float32),
        mesh=mesh,
        compiler_params=pltpu.CompilerParams(use_tc_tiling_on_sc=False),
        scratch_types=[
            pltpu.VMEM((NUM_BLOCKS, BLOCK), jnp.int32),
            pltpu.VMEM((NUM_BLOCKS, BLOCK), jnp.int32),
            pltpu.VMEM((2, BLOCK, EMBED_DIM), jnp.float32),
            pltpu.VMEM_SHARED((NUM_SUBCORES * ROWS_PER_WORKER, EMBED_DIM),
                              jnp.float32),
            pltpu.SemaphoreType.DMA((2,)),
            pltpu.SemaphoreType.DMA((2,)),
        ],
    )
    return pool(idx, scat, table_lin)


def _mlp_kernel(x_ref, len_ref, w1_ref, b1_ref, w2_ref, b2_ref, out_ref):
    x = x_ref[...] / len_ref[...]
    h = jnp.maximum(
        jnp.dot(x, w1_ref[...], preferred_element_type=jnp.float32)
        + b1_ref[...], 0.0)
    out_ref[...] = (
        jnp.dot(h, w2_ref[...], preferred_element_type=jnp.float32)
        + b2_ref[...])


@jax.jit
def _mlp(pooled, text_len, W1, b1, W2, b2):
    bm = 512
    n_hidden = W1.shape[1]
    n_classes = W2.shape[1]
    grid = (BATCH // bm,)
    return pl.pallas_call(
        _mlp_kernel,
        grid=grid,
        in_specs=[
            pl.BlockSpec((bm, EMBED_DIM), lambda i: (i, 0)),
            pl.BlockSpec((bm, 1), lambda i: (i, 0)),
            pl.BlockSpec((EMBED_DIM, n_hidden), lambda i: (0, 0)),
            pl.BlockSpec((1, n_hidden), lambda i: (0, 0)),
            pl.BlockSpec((n_hidden, n_classes), lambda i: (0, 0)),
            pl.BlockSpec((1, n_classes), lambda i: (0, 0)),
        ],
        out_specs=pl.BlockSpec((bm, n_classes), lambda i: (i, 0)),
        out_shape=jax.ShapeDtypeStruct((BATCH, n_classes), jnp.float32),
    )(pooled, text_len.reshape(BATCH, 1), W1, b1.reshape(1, n_hidden),
      W2, b2.reshape(1, n_classes))


def kernel(input_text, text_len, table, W1, b1, W2, b2):
    pooled = _sc_pool(input_text, table)
    return _mlp(pooled, text_len, W1, b1, W2, b2)


# speedup vs baseline: 1.4562x; 1.2966x over previous
import jax
import jax.numpy as jnp
from jax import lax
from jax.experimental import pallas as pl
from jax.experimental.pallas import tpu as pltpu
from jax.experimental.pallas import tpu_sc as plsc

VOCAB = 1000000
EMBED_DIM = 64
PAD_DIM = 128
BATCH = 4096
SEQ = 200

NUM_CORES = 2
NUM_SUBCORES = 16
NUM_WORKERS = NUM_CORES * NUM_SUBCORES
IDX_PER_WORKER = BATCH * SEQ // NUM_WORKERS
ROWS_PER_WORKER = BATCH // NUM_WORKERS
BLOCK = 128
NUM_BLOCKS = IDX_PER_WORKER // BLOCK

CHUNK = 1000
NUM_CHUNKS = VOCAB // CHUNK
CHUNKS_PER_WORKER = 32


TP_CHUNK = 4096


def _transpose_pad_kernel(in_ref, out_ref):
    out_ref[:, :EMBED_DIM] = in_ref[...].T


@jax.jit
def _transpose_pad(table_t):
    grid = (VOCAB // TP_CHUNK,)
    return pl.pallas_call(
        _transpose_pad_kernel,
        grid=grid,
        in_specs=[pl.BlockSpec((EMBED_DIM, TP_CHUNK), lambda j: (0, j))],
        out_specs=pl.BlockSpec((TP_CHUNK, PAD_DIM), lambda j: (j, 0)),
        out_shape=jax.ShapeDtypeStruct((VOCAB, PAD_DIM), jnp.float32),
    )(table_t)


def _sc_pool_kernel(idx_hbm, scat_hbm, table_hbm, out_hbm,
                    idx_v, scat_v, rows_v, acc_sh, gsems, ssems):
    cid = lax.axis_index("c")
    sid = lax.axis_index("s")
    wid = sid * NUM_CORES + cid

    pltpu.sync_copy(idx_hbm.at[wid], idx_v)
    pltpu.sync_copy(scat_hbm.at[sid], scat_v)

    zeros16 = jnp.zeros((16,), jnp.float32)

    @pl.loop(0, BLOCK)
    def _(r):
        @pl.loop(0, EMBED_DIM, step=16)
        def _(c):
            rows_v[0, r, pl.ds(c, 16)] = zeros16

    pltpu.sync_copy(rows_v.at[0],
                    acc_sh.at[pl.ds(sid * ROWS_PER_WORKER, ROWS_PER_WORKER)])

    def gather(b, slot):
        return pltpu.make_async_copy(
            table_hbm.at[idx_v.at[b]], rows_v.at[slot], gsems.at[slot])

    def scatter_start(b, slot):
        pltpu.async_copy(
            rows_v.at[slot], acc_sh.at[scat_v.at[b]], ssems.at[slot],
            add=True)

    def scatter_wait(b, slot):
        pltpu.make_async_copy(
            rows_v.at[slot], acc_sh.at[scat_v.at[b]], ssems.at[slot]).wait()

    gather(0, 0).start()
    gather(1, 1).start()

    @pl.loop(0, NUM_BLOCKS, step=2)
    def _(b):
        gather(b, 0).wait()
        scatter_start(b, 0)
        gather(b + 1, 1).wait()
        scatter_start(b + 1, 1)
        scatter_wait(b, 0)

        @pl.when(b + 2 < NUM_BLOCKS)
        def _():
            gather(b + 2, 0).start()

        scatter_wait(b + 1, 1)

        @pl.when(b + 3 < NUM_BLOCKS)
        def _():
            gather(b + 3, 1).start()

    pltpu.sync_copy(acc_sh.at[pl.ds(sid * ROWS_PER_WORKER, ROWS_PER_WORKER)],
                    out_hbm.at[pl.ds(wid * ROWS_PER_WORKER, ROWS_PER_WORKER)])


@jax.jit
def _sc_pool(input_text, table):
    idx = (input_text.reshape(NUM_WORKERS, NUM_BLOCKS, BLOCK)) * 2
    base = (jnp.arange(IDX_PER_WORKER, dtype=jnp.int32) // SEQ).reshape(
        1, NUM_BLOCKS, BLOCK)
    offs = (jnp.arange(NUM_SUBCORES, dtype=jnp.int32)
            * ROWS_PER_WORKER).reshape(NUM_SUBCORES, 1, 1)
    scat = base + offs

    mesh = plsc.VectorSubcoreMesh(core_axis_name="c", subcore_axis_name="s")

    table_lin = _transpose_pad(table.T).reshape(2 * VOCAB, EMBED_DIM)

    pool = pl.kernel(
        _sc_pool_kernel,
        out_type=jax.ShapeDtypeStruct((BATCH, EMBED_DIM), jnp.float32),
        mesh=mesh,
        compiler_params=pltpu.CompilerParams(use_tc_tiling_on_sc=False),
        scratch_types=[
            pltpu.VMEM((NUM_BLOCKS, BLOCK), jnp.int32),
            pltpu.VMEM((NUM_BLOCKS, BLOCK), jnp.int32),
            pltpu.VMEM((2, BLOCK, EMBED_DIM), jnp.float32),
            pltpu.VMEM_SHARED((NUM_SUBCORES * ROWS_PER_WORKER, EMBED_DIM),
                              jnp.float32),
            pltpu.SemaphoreType.DMA((2,)),
            pltpu.SemaphoreType.DMA((2,)),
        ],
    )
    return pool(idx, scat, table_lin)


def _mlp_kernel(x_ref, len_ref, w1_ref, b1_ref, w2_ref, b2_ref, out_ref):
    x = x_ref[...] / len_ref[...]
    h = jnp.maximum(
        jnp.dot(x, w1_ref[...], preferred_element_type=jnp.float32)
        + b1_ref[...], 0.0)
    out_ref[...] = (
        jnp.dot(h, w2_ref[...], preferred_element_type=jnp.float32)
        + b2_ref[...])


@jax.jit
def _mlp(pooled, text_len, W1, b1, W2, b2):
    bm = 512
    n_hidden = W1.shape[1]
    n_classes = W2.shape[1]
    grid = (BATCH // bm,)
    return pl.pallas_call(
        _mlp_kernel,
        grid=grid,
        in_specs=[
            pl.BlockSpec((bm, EMBED_DIM), lambda i: (i, 0)),
            pl.BlockSpec((bm, 1), lambda i: (i, 0)),
            pl.BlockSpec((EMBED_DIM, n_hidden), lambda i: (0, 0)),
            pl.BlockSpec((1, n_hidden), lambda i: (0, 0)),
            pl.BlockSpec((n_hidden, n_classes), lambda i: (0, 0)),
            pl.BlockSpec((1, n_classes), lambda i: (0, 0)),
        ],
        out_specs=pl.BlockSpec((bm, n_classes), lambda i: (i, 0)),
        out_shape=jax.ShapeDtypeStruct((BATCH, n_classes), jnp.float32),
    )(pooled, text_len.reshape(BATCH, 1), W1, b1.reshape(1, n_hidden),
      W2, b2.reshape(1, n_classes))


def kernel(input_text, text_len, table, W1, b1, W2, b2):
    pooled = _sc_pool(input_text, table)
    return _mlp(pooled, text_len, W1, b1, W2, b2)


# speedup vs baseline: 1.4590x; 1.0019x over previous
import jax
import jax.numpy as jnp
from jax import lax
from jax.experimental import pallas as pl
from jax.experimental.pallas import tpu as pltpu
from jax.experimental.pallas import tpu_sc as plsc

VOCAB = 1000000
EMBED_DIM = 64
PAD_DIM = 128
BATCH = 4096
SEQ = 200

NUM_CORES = 2
NUM_SUBCORES = 16
NUM_WORKERS = NUM_CORES * NUM_SUBCORES
IDX_PER_WORKER = BATCH * SEQ // NUM_WORKERS
ROWS_PER_WORKER = BATCH // NUM_WORKERS
BLOCK = 128
NUM_BLOCKS = IDX_PER_WORKER // BLOCK

CHUNK = 1000
NUM_CHUNKS = VOCAB // CHUNK
CHUNKS_PER_WORKER = 32


TP_CHUNK = 4096


def _transpose_pad_kernel(in_ref, out_ref):
    out_ref[:, :EMBED_DIM] = in_ref[...].T


@jax.jit
def _transpose_pad(table_t):
    grid = (pl.cdiv(VOCAB, TP_CHUNK),)
    return pl.pallas_call(
        _transpose_pad_kernel,
        grid=grid,
        in_specs=[pl.BlockSpec((EMBED_DIM, TP_CHUNK), lambda j: (0, j))],
        out_specs=pl.BlockSpec((TP_CHUNK, PAD_DIM), lambda j: (j, 0)),
        out_shape=jax.ShapeDtypeStruct((VOCAB, PAD_DIM), jnp.float32),
    )(table_t)


def _sc_pool_kernel(idx_hbm, scat_hbm, table_hbm, out_hbm,
                    idx_v, scat_v, rows_v, acc_sh, gsems, ssems):
    cid = lax.axis_index("c")
    sid = lax.axis_index("s")
    wid = sid * NUM_CORES + cid

    pltpu.sync_copy(idx_hbm.at[wid], idx_v)
    pltpu.sync_copy(scat_hbm.at[sid], scat_v)

    zeros16 = jnp.zeros((16,), jnp.float32)

    @pl.loop(0, BLOCK)
    def _(r):
        @pl.loop(0, EMBED_DIM, step=16)
        def _(c):
            rows_v[0, r, pl.ds(c, 16)] = zeros16

    pltpu.sync_copy(rows_v.at[0],
                    acc_sh.at[pl.ds(sid * ROWS_PER_WORKER, ROWS_PER_WORKER)])

    def gather(b, slot):
        return pltpu.make_async_copy(
            table_hbm.at[idx_v.at[b]], rows_v.at[slot], gsems.at[slot])

    def scatter_start(b, slot):
        pltpu.async_copy(
            rows_v.at[slot], acc_sh.at[scat_v.at[b]], ssems.at[slot],
            add=True)

    def scatter_wait(b, slot):
        pltpu.make_async_copy(
            rows_v.at[slot], acc_sh.at[scat_v.at[b]], ssems.at[slot]).wait()

    gather(0, 0).start()
    gather(1, 1).start()

    @pl.loop(0, NUM_BLOCKS, step=2)
    def _(b):
        gather(b, 0).wait()
        scatter_start(b, 0)
        gather(b + 1, 1).wait()
        scatter_start(b + 1, 1)
        scatter_wait(b, 0)

        @pl.when(b + 2 < NUM_BLOCKS)
        def _():
            gather(b + 2, 0).start()

        scatter_wait(b + 1, 1)

        @pl.when(b + 3 < NUM_BLOCKS)
        def _():
            gather(b + 3, 1).start()

    pltpu.sync_copy(acc_sh.at[pl.ds(sid * ROWS_PER_WORKER, ROWS_PER_WORKER)],
                    out_hbm.at[pl.ds(wid * ROWS_PER_WORKER, ROWS_PER_WORKER)])


@jax.jit
def _sc_pool(input_text, table):
    idx = (input_text.reshape(NUM_WORKERS, NUM_BLOCKS, BLOCK)) * 2
    base = (jnp.arange(IDX_PER_WORKER, dtype=jnp.int32) // SEQ).reshape(
        1, NUM_BLOCKS, BLOCK)
    offs = (jnp.arange(NUM_SUBCORES, dtype=jnp.int32)
            * ROWS_PER_WORKER).reshape(NUM_SUBCORES, 1, 1)
    scat = base + offs

    mesh = plsc.VectorSubcoreMesh(core_axis_name="c", subcore_axis_name="s")

    table_lin = _transpose_pad(table.T).reshape(2 * VOCAB, EMBED_DIM)

    pool = pl.kernel(
        _sc_pool_kernel,
        out_type=jax.ShapeDtypeStruct((BATCH, EMBED_DIM), jnp.float32),
        mesh=mesh,
        compiler_params=pltpu.CompilerParams(use_tc_tiling_on_sc=False),
        scratch_types=[
            pltpu.VMEM((NUM_BLOCKS, BLOCK), jnp.int32),
            pltpu.VMEM((NUM_BLOCKS, BLOCK), jnp.int32),
            pltpu.VMEM((2, BLOCK, EMBED_DIM), jnp.float32),
            pltpu.VMEM_SHARED((NUM_SUBCORES * ROWS_PER_WORKER, EMBED_DIM),
                              jnp.float32),
            pltpu.SemaphoreType.DMA((2,)),
            pltpu.SemaphoreType.DMA((2,)),
        ],
    )
    return pool(idx, scat, table_lin)


def _mlp_kernel(x_ref, len_ref, w1_ref, b1_ref, w2_ref, b2_ref, out_ref):
    x = x_ref[...] / len_ref[...]
    h = jnp.maximum(
        jnp.dot(x, w1_ref[...], preferred_element_type=jnp.float32)
        + b1_ref[...], 0.0)
    out_ref[...] = (
        jnp.dot(h, w2_ref[...], preferred_element_type=jnp.float32)
        + b2_ref[...])


@jax.jit
def _mlp(pooled, text_len, W1, b1, W2, b2):
    bm = 512
    n_hidden = W1.shape[1]
    n_classes = W2.shape[1]
    grid = (BATCH // bm,)
    return pl.pallas_call(
        _mlp_kernel,
        grid=grid,
        in_specs=[
            pl.BlockSpec((bm, EMBED_DIM), lambda i: (i, 0)),
            pl.BlockSpec((bm, 1), lambda i: (i, 0)),
            pl.BlockSpec((EMBED_DIM, n_hidden), lambda i: (0, 0)),
            pl.BlockSpec((1, n_hidden), lambda i: (0, 0)),
            pl.BlockSpec((n_hidden, n_classes), lambda i: (0, 0)),
            pl.BlockSpec((1, n_classes), lambda i: (0, 0)),
        ],
        out_specs=pl.BlockSpec((bm, n_classes), lambda i: (i, 0)),
        out_shape=jax.ShapeDtypeStruct((BATCH, n_classes), jnp.float32),
    )(pooled, text_len.reshape(BATCH, 1), W1, b1.reshape(1, n_hidden),
      W2, b2.reshape(1, n_classes))


def kernel(input_text, text_len, table, W1, b1, W2, b2):
    pooled = _sc_pool(input_text, table)
    return _mlp(pooled, text_len, W1, b1, W2, b2)
